# baseline (speedup 1.0000x reference)
import functools

import jax
import jax.numpy as jnp
from jax import lax
from jax.experimental import pallas as pl
from jax.experimental.pallas import tpu as pltpu
from jax.experimental.pallas import tpu_sc as plsc

N = 10000
E = 320000
G = 256

NCORE = 2
NSUB = 16
E_PER_CORE = E // NCORE
E_PER_SUB = E_PER_CORE // NSUB
NSTEP = E_PER_SUB // 128
REM = E_PER_SUB - NSTEP * 128
ROWS_MAIN = 624
ROWS_TAIL_OFF = ROWS_MAIN * NSUB
ROWS_TAIL = N - ROWS_TAIL_OFF

f32 = jnp.float32
i32 = jnp.int32


def _sc_mesh():
    return plsc.VectorSubcoreMesh(core_axis_name="c", subcore_axis_name="s",
                                  num_cores=NCORE, num_subcores=NSUB)


def _make_deg_kernel():
    @functools.partial(
        pl.kernel,
        out_type=jax.ShapeDtypeStruct((NCORE, N, 128), f32),
        mesh=_sc_mesh(),
        scratch_types=[
            pltpu.VMEM_SHARED((N, 128), f32),
            pltpu.VMEM((128, 128), f32),
            pltpu.VMEM((1, 128), i32),
            pltpu.VMEM((1, 16), i32),
        ],
    )
    def deg_kernel(dst_hbm, ones_hbm, zeros_hbm, degp_hbm, acc, ones_v, didx,
                   didx16):
        c = lax.axis_index("c")
        s = lax.axis_index("s")
        r0 = s * ROWS_MAIN
        pltpu.sync_copy(ones_hbm, ones_v)
        pltpu.sync_copy(zeros_hbm.at[pl.ds(r0, ROWS_MAIN)],
                        acc.at[pl.ds(r0, ROWS_MAIN)])

        @pl.when(s == NSUB - 1)
        def _():
            pltpu.sync_copy(zeros_hbm.at[pl.ds(ROWS_TAIL_OFF, ROWS_TAIL)],
                            acc.at[pl.ds(ROWS_TAIL_OFF, ROWS_TAIL)])

        plsc.subcore_barrier()
        base = c * E_PER_CORE + s * E_PER_SUB

        def step(i, carry):
            off = base + i * 128
            pltpu.sync_copy(dst_hbm.at[pl.ds(off, 128)], didx.at[0])
            pltpu.sync_copy(ones_v, acc.at[didx.at[0]], add=True)
            return carry

        lax.fori_loop(0, NSTEP, step, 0)
        off = base + NSTEP * 128
        pltpu.sync_copy(dst_hbm.at[pl.ds(off, REM)], didx16.at[0])
        pltpu.sync_copy(ones_v.at[pl.ds(0, REM)], acc.at[didx16.at[0]],
                        add=True)
        plsc.subcore_barrier()
        pltpu.sync_copy(acc.at[pl.ds(r0, ROWS_MAIN)],
                        degp_hbm.at[c, pl.ds(r0, ROWS_MAIN)])

        @pl.when(s == NSUB - 1)
        def _():
            pltpu.sync_copy(acc.at[pl.ds(ROWS_TAIL_OFF, ROWS_TAIL)],
                            degp_hbm.at[c, pl.ds(ROWS_TAIL_OFF, ROWS_TAIL)])

    return deg_kernel


EB64 = 64
NSTEPB = E_PER_SUB // EB64
M = 4
KLEAD = 2
NOUT = NSTEPB // M


def _make_scatter_kernel(C):
    scratch = [pltpu.VMEM_SHARED((N, 128), f32)]
    scratch += [pltpu.VMEM((1, EB64), i32) for _ in range(M)]
    scratch += [pltpu.VMEM((1, EB64), i32) for _ in range(M)]
    scratch += [pltpu.VMEM((EB64, 128), f32) for _ in range(M)]
    scratch += [
        pltpu.VMEM((1, 16), i32),
        pltpu.VMEM((1, 16), i32),
        pltpu.VMEM((16, 128), f32),
    ]
    scratch += [pltpu.SemaphoreType.DMA for _ in range(2 * M)]

    @functools.partial(
        pl.kernel,
        out_type=jax.ShapeDtypeStruct((NCORE * C, N, 128), f32),
        mesh=_sc_mesh(),
        scratch_types=scratch,
    )
    def scatter_kernel(*refs):
        src_hbm, dst_hbm = refs[0], refs[1]
        ys = refs[2:2 + C]
        zeros_hbm = refs[2 + C]
        z_hbm = refs[3 + C]
        sc = refs[4 + C:]
        acc = sc[0]
        sidx = sc[1:1 + M]
        didx = sc[1 + M:1 + 2 * M]
        rows = sc[1 + 2 * M:1 + 3 * M]
        sidx16, didx16, rows16 = sc[1 + 3 * M:4 + 3 * M]
        gsem = sc[4 + 3 * M:4 + 4 * M]
        ssem = sc[4 + 4 * M:4 + 5 * M]

        c = lax.axis_index("c")
        s = lax.axis_index("s")
        r0 = s * ROWS_MAIN
        base = c * E_PER_CORE + s * E_PER_SUB

        for chunk in range(C):
            y = ys[chunk]
            pltpu.sync_copy(zeros_hbm.at[pl.ds(r0, ROWS_MAIN)],
                            acc.at[pl.ds(r0, ROWS_MAIN)])

            @pl.when(s == NSUB - 1)
            def _():
                pltpu.sync_copy(zeros_hbm.at[pl.ds(ROWS_TAIL_OFF, ROWS_TAIL)],
                                acc.at[pl.ds(ROWS_TAIL_OFF, ROWS_TAIL)])

            plsc.subcore_barrier()

            for j in range(KLEAD):
                pltpu.sync_copy(src_hbm.at[pl.ds(base + j * EB64, EB64)],
                                sidx[j].at[0])
                pltpu.async_copy(y.at[sidx[j].at[0]], rows[j], gsem[j])

            def outer(o, carry):
                for j in range(M):
                    v_off = base + (o * M + j) * EB64
                    br = (j + KLEAD) % M
                    vr_off = v_off + KLEAD * EB64

                    def refill(br=br, vr_off=vr_off, guard_wait=True):
                        if guard_wait:
                            pltpu.make_async_copy(
                                rows[br], acc.at[didx[br].at[0]],
                                ssem[br]).wait()
                        pltpu.sync_copy(src_hbm.at[pl.ds(vr_off, EB64)],
                                        sidx[br].at[0])
                        pltpu.async_copy(y.at[sidx[br].at[0]], rows[br],
                                         gsem[br])

                    if j < KLEAD:
                        @pl.when(o > 0)
                        def _(br=br):
                            pltpu.make_async_copy(
                                rows[br], acc.at[didx[br].at[0]],
                                ssem[br]).wait()
                        refill(guard_wait=False)
                    else:
                        @pl.when(o < NOUT - 1)
                        def _(refill=refill):
                            refill()

                    pltpu.make_async_copy(y.at[sidx[j].at[0]], rows[j],
                                          gsem[j]).wait()
                    pltpu.sync_copy(dst_hbm.at[pl.ds(v_off, EB64)],
                                    didx[j].at[0])
                    pltpu.async_copy(rows[j], acc.at[didx[j].at[0]], ssem[j],
                                     add=True)
                return carry

            lax.fori_loop(0, NOUT, outer, 0)
            for j in range(M):
                pltpu.make_async_copy(rows[j], acc.at[didx[j].at[0]],
                                      ssem[j]).wait()
            off = base + NSTEPB * EB64
            pltpu.sync_copy(src_hbm.at[pl.ds(off, REM)], sidx16.at[0])
            pltpu.sync_copy(dst_hbm.at[pl.ds(off, REM)], didx16.at[0])
            pltpu.async_copy(y.at[sidx16.at[0]], rows16, gsem[0]).wait()
            pltpu.sync_copy(rows16, acc.at[didx16.at[0]], add=True)
            plsc.subcore_barrier()
            zi = c * C + chunk
            pltpu.sync_copy(acc.at[pl.ds(r0, ROWS_MAIN)],
                            z_hbm.at[zi, pl.ds(r0, ROWS_MAIN)])

            @pl.when(s == NSUB - 1)
            def _():
                pltpu.sync_copy(acc.at[pl.ds(ROWS_TAIL_OFF, ROWS_TAIL)],
                                z_hbm.at[zi, pl.ds(ROWS_TAIL_OFF, ROWS_TAIL)])

            plsc.subcore_barrier()

    return scatter_kernel


BLK = 400
NBLK = N // BLK


def _tck1(x, dega, degb, batchcol, W1):
    def body(x_ref, da_ref, db_ref, b_ref, w_ref, y_ref, dinv_ref, cnt_ref):
        i = pl.program_id(0)
        deg = da_ref[:, 0:1] + db_ref[:, 0:1] + 1.0
        dinv = lax.rsqrt(deg)
        dinv_ref[...] = dinv
        y_ref[...] = dinv * jnp.dot(x_ref[...], w_ref[...],
                                    preferred_element_type=f32)
        oh = (b_ref[...] == lax.broadcasted_iota(i32, (1, G), 1)).astype(f32)
        csum = jnp.sum(oh, axis=0, keepdims=True)

        @pl.when(i == 0)
        def _():
            cnt_ref[...] = csum

        @pl.when(i > 0)
        def _():
            cnt_ref[...] += csum

    return pl.pallas_call(
        body,
        grid=(NBLK,),
        in_specs=[
            pl.BlockSpec((BLK, 128), lambda i: (i, 0)),
            pl.BlockSpec((BLK, 128), lambda i: (i, 0)),
            pl.BlockSpec((BLK, 128), lambda i: (i, 0)),
            pl.BlockSpec((BLK, 1), lambda i: (i, 0)),
            pl.BlockSpec((128, 128), lambda i: (0, 0)),
        ],
        out_specs=[
            pl.BlockSpec((BLK, 128), lambda i: (i, 0)),
            pl.BlockSpec((BLK, 1), lambda i: (i, 0)),
            pl.BlockSpec((1, G), lambda i: (0, 0)),
        ],
        out_shape=[
            jax.ShapeDtypeStruct((N, 128), f32),
            jax.ShapeDtypeStruct((N, 1), f32),
            jax.ShapeDtypeStruct((1, G), f32),
        ],
    )(x, dega, degb, batchcol, W1)


def _tck2(z1a, z1b, y1, dinv, b1, W2):
    def body(za_ref, zb_ref, y_ref, d_ref, b_ref, w_ref, o0_ref, o1_ref):
        d = d_ref[...]
        h = jnp.maximum(d * (za_ref[...] + zb_ref[...] + y_ref[...])
                        + b_ref[...], 0.0)
        y2 = d * jnp.dot(h, w_ref[...], preferred_element_type=f32)
        o0_ref[...] = y2[:, :128]
        o1_ref[...] = y2[:, 128:]

    return pl.pallas_call(
        body,
        grid=(NBLK,),
        in_specs=[
            pl.BlockSpec((BLK, 128), lambda i: (i, 0)),
            pl.BlockSpec((BLK, 128), lambda i: (i, 0)),
            pl.BlockSpec((BLK, 128), lambda i: (i, 0)),
            pl.BlockSpec((BLK, 1), lambda i: (i, 0)),
            pl.BlockSpec((1, 128), lambda i: (0, 0)),
            pl.BlockSpec((128, 256), lambda i: (0, 0)),
        ],
        out_specs=[
            pl.BlockSpec((BLK, 128), lambda i: (i, 0)),
            pl.BlockSpec((BLK, 128), lambda i: (i, 0)),
        ],
        out_shape=[
            jax.ShapeDtypeStruct((N, 128), f32),
            jax.ShapeDtypeStruct((N, 128), f32),
        ],
    )(z1a, z1b, y1, dinv, b1, W2)


def _tck3(za0, zb0, za1, zb1, y20, y21, dinv, b2, W3):
    def body(za0_ref, zb0_ref, za1_ref, zb1_ref, y0_ref, y1_ref, d_ref,
             b_ref, w_ref, o0_ref, o1_ref, o2_ref, o3_ref):
        d = d_ref[...]
        z = jnp.concatenate(
            [za0_ref[...] + zb0_ref[...] + y0_ref[...],
             za1_ref[...] + zb1_ref[...] + y1_ref[...]], axis=1)
        h = jnp.maximum(d * z + b_ref[...], 0.0)
        y3 = d * jnp.dot(h, w_ref[...], preferred_element_type=f32)
        o0_ref[...] = y3[:, 0:128]
        o1_ref[...] = y3[:, 128:256]
        o2_ref[...] = y3[:, 256:384]
        o3_ref[...] = y3[:, 384:512]

    blk = lambda w: pl.BlockSpec((BLK, w), lambda i: (i, 0))
    return pl.pallas_call(
        body,
        grid=(NBLK,),
        in_specs=[
            blk(128), blk(128), blk(128), blk(128), blk(128), blk(128),
            blk(1),
            pl.BlockSpec((1, 256), lambda i: (0, 0)),
            pl.BlockSpec((256, 512), lambda i: (0, 0)),
        ],
        out_specs=[blk(128), blk(128), blk(128), blk(128)],
        out_shape=[jax.ShapeDtypeStruct((N, 128), f32)] * 4,
    )(za0, zb0, za1, zb1, y20, y21, dinv, b2, W3)


def _tck4(zs, ys, dinv, b3):
    def body(*refs):
        za = refs[0:4]
        zb = refs[4:8]
        y = refs[8:12]
        d_ref, b_ref, h_ref = refs[12], refs[13], refs[14]
        d = d_ref[...]
        z = jnp.concatenate(
            [za[c][...] + zb[c][...] + y[c][...] for c in range(4)], axis=1)
        h_ref[...] = jnp.maximum(d * z + b_ref[...], 0.0)

    blk = lambda w: pl.BlockSpec((BLK, w), lambda i: (i, 0))
    return pl.pallas_call(
        body,
        grid=(NBLK,),
        in_specs=[blk(128)] * 12 + [
            blk(1),
            pl.BlockSpec((1, 512), lambda i: (0, 0)),
        ],
        out_specs=pl.BlockSpec((BLK, 512), lambda i: (i, 0)),
        out_shape=jax.ShapeDtypeStruct((N, 512), f32),
    )(*zs[0:4], *zs[4:8], *ys, dinv, b3)


def _tck5(seg, h3, batchcol, counts2, Wf1, bf1, Wf2, bf2):
    def body(seg_ref, h_ref, b_ref, cnt_ref, wf1_ref, bf1_ref, wf2_ref,
             bf2_ref, out_ref, mx_ref, sum_ref):
        i = pl.program_id(0)

        @pl.when(i == 0)
        def _():
            mx_ref[...] = jnp.full((G, 512), -1e30, f32)
            sum_ref[...] = jnp.zeros((G, 512), f32)

        h = h_ref[...]
        b = b_ref[...]
        oh = (b == lax.broadcasted_iota(i32, (1, G), 1)).astype(f32)
        sum_ref[...] += lax.dot_general(oh, h, (((0,), (0,)), ((), ())),
                                        preferred_element_type=f32)
        lo = seg_ref[2 * i]
        hi = seg_ref[2 * i + 1]

        def mbody(g, carry):
            m = b == g
            cand = jnp.max(jnp.where(m, h, -1e30), axis=0, keepdims=True)
            cur = mx_ref[pl.ds(g, 1), :]
            mx_ref[pl.ds(g, 1), :] = jnp.maximum(cur, cand)
            return carry

        lax.fori_loop(lo, hi + 1, mbody, 0)

        @pl.when(i == NBLK - 1)
        def _():
            cnt = cnt_ref[...]
            mean = sum_ref[...] / jnp.maximum(cnt, 1.0)
            mx = jnp.where(cnt > 0, mx_ref[...], 0.0)
            pooled = jnp.concatenate([mx, mean], axis=1)
            zf = jnp.maximum(
                jnp.dot(pooled, wf1_ref[...], preferred_element_type=f32)
                + bf1_ref[...], 0.0)
            out_ref[...] = (jnp.dot(zf, wf2_ref[...],
                                    preferred_element_type=f32) + bf2_ref[...])

    grid_spec = pltpu.PrefetchScalarGridSpec(
        num_scalar_prefetch=1,
        grid=(NBLK,),
        in_specs=[
            pl.BlockSpec((BLK, 512), lambda i, s: (i, 0)),
            pl.BlockSpec((BLK, 1), lambda i, s: (i, 0)),
            pl.BlockSpec((G, 1), lambda i, s: (0, 0)),
            pl.BlockSpec((1024, 1024), lambda i, s: (0, 0)),
            pl.BlockSpec((1, 1024), lambda i, s: (0, 0)),
            pl.BlockSpec((1024, 128), lambda i, s: (0, 0)),
            pl.BlockSpec((1, 128), lambda i, s: (0, 0)),
        ],
        out_specs=pl.BlockSpec((G, 128), lambda i, s: (0, 0)),
        scratch_shapes=[
            pltpu.VMEM((G, 512), f32),
            pltpu.VMEM((G, 512), f32),
        ],
    )
    return pl.pallas_call(
        body,
        grid_spec=grid_spec,
        out_shape=jax.ShapeDtypeStruct((G, 128), f32),
    )(seg, h3, batchcol, counts2, Wf1, bf1, Wf2, bf2)


_SC_KERNELS = {}


def _deg_kernel(*args):
    if "deg" not in _SC_KERNELS:
        _SC_KERNELS["deg"] = _make_deg_kernel()
    return _SC_KERNELS["deg"](*args)


def _scatter(C, *args):
    if C not in _SC_KERNELS:
        _SC_KERNELS[C] = _make_scatter_kernel(C)
    return _SC_KERNELS[C](*args)


def _scatter1(*args):
    return _scatter(1, *args)


def _scatter2(*args):
    return _scatter(2, *args)


def _scatter4(*args):
    return _scatter(4, *args)


def kernel(x, edge_index, batch, W1, b1, W2, b2, W3, b3, Wf1, bf1, Wf2, bf2):
    src = edge_index[0]
    dst = edge_index[1]
    zeros_n128 = jnp.zeros((N, 128), f32)
    ones_n128 = jnp.ones((128, 128), f32)
    batchcol = batch.reshape(N, 1)

    degp = _deg_kernel(dst, ones_n128, zeros_n128)
    y1, dinv, counts = _tck1(x, degp[0], degp[1], batchcol, W1)

    z1 = _scatter1(src, dst, y1, zeros_n128)
    y20, y21 = _tck2(z1[0], z1[1], y1, dinv, b1.reshape(1, 128), W2)

    z2 = _scatter2(src, dst, y20, y21, zeros_n128)
    y3c = _tck3(z2[0], z2[2], z2[1], z2[3], y20, y21, dinv,
                b2.reshape(1, 256), W3)

    z3 = _scatter4(src, dst, *y3c, zeros_n128)
    h3 = _tck4([z3[c] for c in range(8)], list(y3c), dinv,
               b3.reshape(1, 512))

    seg = jnp.stack([batch[0::BLK], batch[BLK - 1::BLK]], axis=1).reshape(-1)
    counts2 = counts.reshape(G, 1)
    out = _tck5(seg, h3, batchcol, counts2, Wf1, bf1.reshape(1, 1024),
                Wf2, bf2.reshape(1, 128))
    return out


# speedup vs baseline: 1.1939x; 1.1939x over previous
import functools

import jax
import jax.numpy as jnp
from jax import lax
from jax.experimental import pallas as pl
from jax.experimental.pallas import tpu as pltpu
from jax.experimental.pallas import tpu_sc as plsc

N = 10000
E = 320000
G = 256

NCORE = 2
NSUB = 16
E_PER_CORE = E // NCORE
E_PER_SUB = E_PER_CORE // NSUB
NSTEP = E_PER_SUB // 128
REM = E_PER_SUB - NSTEP * 128
ROWS_MAIN = 624
ROWS_TAIL_OFF = ROWS_MAIN * NSUB
ROWS_TAIL = N - ROWS_TAIL_OFF

f32 = jnp.float32
i32 = jnp.int32


def _sc_mesh():
    return plsc.VectorSubcoreMesh(core_axis_name="c", subcore_axis_name="s",
                                  num_cores=NCORE, num_subcores=NSUB)


def _make_deg_kernel():
    @functools.partial(
        pl.kernel,
        out_type=jax.ShapeDtypeStruct((NCORE, N, 128), f32),
        mesh=_sc_mesh(),
        scratch_types=[
            pltpu.VMEM_SHARED((N, 128), f32),
            pltpu.VMEM((128, 128), f32),
            pltpu.VMEM((1, 128), i32),
            pltpu.VMEM((1, 16), i32),
        ],
    )
    def deg_kernel(dst_hbm, ones_hbm, zeros_hbm, degp_hbm, acc, ones_v, didx,
                   didx16):
        c = lax.axis_index("c")
        s = lax.axis_index("s")
        r0 = s * ROWS_MAIN
        pltpu.sync_copy(ones_hbm, ones_v)
        pltpu.sync_copy(zeros_hbm.at[pl.ds(r0, ROWS_MAIN)],
                        acc.at[pl.ds(r0, ROWS_MAIN)])

        @pl.when(s == NSUB - 1)
        def _():
            pltpu.sync_copy(zeros_hbm.at[pl.ds(ROWS_TAIL_OFF, ROWS_TAIL)],
                            acc.at[pl.ds(ROWS_TAIL_OFF, ROWS_TAIL)])

        plsc.subcore_barrier()
        base = c * E_PER_CORE + s * E_PER_SUB

        def step(i, carry):
            off = base + i * 128
            pltpu.sync_copy(dst_hbm.at[pl.ds(off, 128)], didx.at[0])
            pltpu.sync_copy(ones_v, acc.at[didx.at[0]], add=True)
            return carry

        lax.fori_loop(0, NSTEP, step, 0)
        off = base + NSTEP * 128
        pltpu.sync_copy(dst_hbm.at[pl.ds(off, REM)], didx16.at[0])
        pltpu.sync_copy(ones_v.at[pl.ds(0, REM)], acc.at[didx16.at[0]],
                        add=True)
        plsc.subcore_barrier()
        pltpu.sync_copy(acc.at[pl.ds(r0, ROWS_MAIN)],
                        degp_hbm.at[c, pl.ds(r0, ROWS_MAIN)])

        @pl.when(s == NSUB - 1)
        def _():
            pltpu.sync_copy(acc.at[pl.ds(ROWS_TAIL_OFF, ROWS_TAIL)],
                            degp_hbm.at[c, pl.ds(ROWS_TAIL_OFF, ROWS_TAIL)])

    return deg_kernel


EB = E // 128
B_PER_CORE = EB // NCORE
NP = 39


def _make_scatter_kernel(C):
    scratch = [
        pltpu.VMEM_SHARED((N, 128), f32),
        pltpu.VMEM((2, 128), i32),
        pltpu.VMEM((2, 128), i32),
        pltpu.VMEM((128, 128), f32),
        pltpu.VMEM((128, 128), f32),
        pltpu.SemaphoreType.DMA,
        pltpu.SemaphoreType.DMA,
        pltpu.SemaphoreType.DMA,
        pltpu.SemaphoreType.DMA,
    ]

    @functools.partial(
        pl.kernel,
        out_type=jax.ShapeDtypeStruct((NCORE * C, N, 128), f32),
        mesh=_sc_mesh(),
        scratch_types=scratch,
    )
    def scatter_kernel(*refs):
        idx_hbm = refs[0]
        ys = refs[1:1 + C]
        zeros_hbm = refs[1 + C]
        z_hbm = refs[2 + C]
        (acc, idxa, idxb, rowsa, rowsb, sema, semb, semsa, semsb) = \
            refs[3 + C:]

        c = lax.axis_index("c")
        s = lax.axis_index("s")
        r0 = s * ROWS_MAIN
        b0 = c * B_PER_CORE + 78 * s + jnp.minimum(s, 2)

        for chunk in range(C):
            y = ys[chunk]
            pltpu.sync_copy(zeros_hbm.at[pl.ds(r0, ROWS_MAIN)],
                            acc.at[pl.ds(r0, ROWS_MAIN)])

            @pl.when(s == NSUB - 1)
            def _():
                pltpu.sync_copy(zeros_hbm.at[pl.ds(ROWS_TAIL_OFF, ROWS_TAIL)],
                                acc.at[pl.ds(ROWS_TAIL_OFF, ROWS_TAIL)])

            plsc.subcore_barrier()

            pltpu.sync_copy(idx_hbm.at[b0], idxa)
            pltpu.async_copy(y.at[idxa.at[0]], rowsa, sema)

            def pair(p, carry):
                ba = b0 + 2 * p
                bb = ba + 1

                @pl.when(p > 0)
                def _():
                    pltpu.make_async_copy(rowsb, acc.at[idxb.at[1]],
                                          semsb).wait()

                pltpu.sync_copy(idx_hbm.at[bb], idxb)
                pltpu.async_copy(y.at[idxb.at[0]], rowsb, semb)
                pltpu.make_async_copy(y.at[idxa.at[0]], rowsa, sema).wait()
                pltpu.async_copy(rowsa, acc.at[idxa.at[1]], semsa, add=True)

                @pl.when(p < NP - 1)
                def _():
                    pltpu.make_async_copy(rowsa, acc.at[idxa.at[1]],
                                          semsa).wait()
                    pltpu.sync_copy(idx_hbm.at[ba + 2], idxa)
                    pltpu.async_copy(y.at[idxa.at[0]], rowsa, sema)

                pltpu.make_async_copy(y.at[idxb.at[0]], rowsb, semb).wait()
                pltpu.async_copy(rowsb, acc.at[idxb.at[1]], semsb, add=True)
                return carry

            lax.fori_loop(0, NP, pair, 0)
            pltpu.make_async_copy(rowsa, acc.at[idxa.at[1]], semsa).wait()
            pltpu.make_async_copy(rowsb, acc.at[idxb.at[1]], semsb).wait()

            @pl.when(s < 2)
            def _():
                pltpu.sync_copy(idx_hbm.at[b0 + 2 * NP], idxa)
                pltpu.async_copy(y.at[idxa.at[0]], rowsa, sema).wait()
                pltpu.sync_copy(rowsa, acc.at[idxa.at[1]], add=True)

            plsc.subcore_barrier()
            zi = c * C + chunk
            pltpu.sync_copy(acc.at[pl.ds(r0, ROWS_MAIN)],
                            z_hbm.at[zi, pl.ds(r0, ROWS_MAIN)])

            @pl.when(s == NSUB - 1)
            def _():
                pltpu.sync_copy(acc.at[pl.ds(ROWS_TAIL_OFF, ROWS_TAIL)],
                                z_hbm.at[zi, pl.ds(ROWS_TAIL_OFF, ROWS_TAIL)])

            plsc.subcore_barrier()

    return scatter_kernel


BLK = 400
NBLK = N // BLK


def _tck1(x, dega, degb, batchcol, W1):
    def body(x_ref, da_ref, db_ref, b_ref, w_ref, y_ref, dinv_ref, cnt_ref):
        i = pl.program_id(0)
        deg = da_ref[:, 0:1] + db_ref[:, 0:1] + 1.0
        dinv = lax.rsqrt(deg)
        dinv_ref[...] = dinv
        y_ref[...] = dinv * jnp.dot(x_ref[...], w_ref[...],
                                    preferred_element_type=f32)
        oh = (b_ref[...] == lax.broadcasted_iota(i32, (1, G), 1)).astype(f32)
        csum = jnp.sum(oh, axis=0, keepdims=True)

        @pl.when(i == 0)
        def _():
            cnt_ref[...] = csum

        @pl.when(i > 0)
        def _():
            cnt_ref[...] += csum

    return pl.pallas_call(
        body,
        grid=(NBLK,),
        in_specs=[
            pl.BlockSpec((BLK, 128), lambda i: (i, 0)),
            pl.BlockSpec((BLK, 128), lambda i: (i, 0)),
            pl.BlockSpec((BLK, 128), lambda i: (i, 0)),
            pl.BlockSpec((BLK, 1), lambda i: (i, 0)),
            pl.BlockSpec((128, 128), lambda i: (0, 0)),
        ],
        out_specs=[
            pl.BlockSpec((BLK, 128), lambda i: (i, 0)),
            pl.BlockSpec((BLK, 1), lambda i: (i, 0)),
            pl.BlockSpec((1, G), lambda i: (0, 0)),
        ],
        out_shape=[
            jax.ShapeDtypeStruct((N, 128), f32),
            jax.ShapeDtypeStruct((N, 1), f32),
            jax.ShapeDtypeStruct((1, G), f32),
        ],
    )(x, dega, degb, batchcol, W1)


def _tck2(z1a, z1b, y1, dinv, b1, W2):
    def body(za_ref, zb_ref, y_ref, d_ref, b_ref, w_ref, o0_ref, o1_ref):
        d = d_ref[...]
        h = jnp.maximum(d * (za_ref[...] + zb_ref[...] + y_ref[...])
                        + b_ref[...], 0.0)
        y2 = d * jnp.dot(h, w_ref[...], preferred_element_type=f32)
        o0_ref[...] = y2[:, :128]
        o1_ref[...] = y2[:, 128:]

    return pl.pallas_call(
        body,
        grid=(NBLK,),
        in_specs=[
            pl.BlockSpec((BLK, 128), lambda i: (i, 0)),
            pl.BlockSpec((BLK, 128), lambda i: (i, 0)),
            pl.BlockSpec((BLK, 128), lambda i: (i, 0)),
            pl.BlockSpec((BLK, 1), lambda i: (i, 0)),
            pl.BlockSpec((1, 128), lambda i: (0, 0)),
            pl.BlockSpec((128, 256), lambda i: (0, 0)),
        ],
        out_specs=[
            pl.BlockSpec((BLK, 128), lambda i: (i, 0)),
            pl.BlockSpec((BLK, 128), lambda i: (i, 0)),
        ],
        out_shape=[
            jax.ShapeDtypeStruct((N, 128), f32),
            jax.ShapeDtypeStruct((N, 128), f32),
        ],
    )(z1a, z1b, y1, dinv, b1, W2)


def _tck3(za0, zb0, za1, zb1, y20, y21, dinv, b2, W3):
    def body(za0_ref, zb0_ref, za1_ref, zb1_ref, y0_ref, y1_ref, d_ref,
             b_ref, w_ref, o0_ref, o1_ref, o2_ref, o3_ref):
        d = d_ref[...]
        z = jnp.concatenate(
            [za0_ref[...] + zb0_ref[...] + y0_ref[...],
             za1_ref[...] + zb1_ref[...] + y1_ref[...]], axis=1)
        h = jnp.maximum(d * z + b_ref[...], 0.0)
        y3 = d * jnp.dot(h, w_ref[...], preferred_element_type=f32)
        o0_ref[...] = y3[:, 0:128]
        o1_ref[...] = y3[:, 128:256]
        o2_ref[...] = y3[:, 256:384]
        o3_ref[...] = y3[:, 384:512]

    blk = lambda w: pl.BlockSpec((BLK, w), lambda i: (i, 0))
    return pl.pallas_call(
        body,
        grid=(NBLK,),
        in_specs=[
            blk(128), blk(128), blk(128), blk(128), blk(128), blk(128),
            blk(1),
            pl.BlockSpec((1, 256), lambda i: (0, 0)),
            pl.BlockSpec((256, 512), lambda i: (0, 0)),
        ],
        out_specs=[blk(128), blk(128), blk(128), blk(128)],
        out_shape=[jax.ShapeDtypeStruct((N, 128), f32)] * 4,
    )(za0, zb0, za1, zb1, y20, y21, dinv, b2, W3)


def _tck4(zs, ys, dinv, b3):
    def body(*refs):
        za = refs[0:4]
        zb = refs[4:8]
        y = refs[8:12]
        d_ref, b_ref, h_ref = refs[12], refs[13], refs[14]
        d = d_ref[...]
        z = jnp.concatenate(
            [za[c][...] + zb[c][...] + y[c][...] for c in range(4)], axis=1)
        h_ref[...] = jnp.maximum(d * z + b_ref[...], 0.0)

    blk = lambda w: pl.BlockSpec((BLK, w), lambda i: (i, 0))
    return pl.pallas_call(
        body,
        grid=(NBLK,),
        in_specs=[blk(128)] * 12 + [
            blk(1),
            pl.BlockSpec((1, 512), lambda i: (0, 0)),
        ],
        out_specs=pl.BlockSpec((BLK, 512), lambda i: (i, 0)),
        out_shape=jax.ShapeDtypeStruct((N, 512), f32),
    )(*zs[0:4], *zs[4:8], *ys, dinv, b3)


def _tck5(seg, h3, batchcol, counts2, Wf1, bf1, Wf2, bf2):
    def body(seg_ref, h_ref, b_ref, cnt_ref, wf1_ref, bf1_ref, wf2_ref,
             bf2_ref, out_ref, mx_ref, sum_ref):
        i = pl.program_id(0)

        @pl.when(i == 0)
        def _():
            mx_ref[...] = jnp.full((G, 512), -1e30, f32)
            sum_ref[...] = jnp.zeros((G, 512), f32)

        h = h_ref[...]
        b = b_ref[...]
        oh = (b == lax.broadcasted_iota(i32, (1, G), 1)).astype(f32)
        sum_ref[...] += lax.dot_general(oh, h, (((0,), (0,)), ((), ())),
                                        preferred_element_type=f32)
        lo = seg_ref[2 * i]
        hi = seg_ref[2 * i + 1]

        def mbody(g, carry):
            m = b == g
            cand = jnp.max(jnp.where(m, h, -1e30), axis=0, keepdims=True)
            cur = mx_ref[pl.ds(g, 1), :]
            mx_ref[pl.ds(g, 1), :] = jnp.maximum(cur, cand)
            return carry

        lax.fori_loop(lo, hi + 1, mbody, 0)

        @pl.when(i == NBLK - 1)
        def _():
            cnt = cnt_ref[...]
            mean = sum_ref[...] / jnp.maximum(cnt, 1.0)
            mx = jnp.where(cnt > 0, mx_ref[...], 0.0)
            pooled = jnp.concatenate([mx, mean], axis=1)
            zf = jnp.maximum(
                jnp.dot(pooled, wf1_ref[...], preferred_element_type=f32)
                + bf1_ref[...], 0.0)
            out_ref[...] = (jnp.dot(zf, wf2_ref[...],
                                    preferred_element_type=f32) + bf2_ref[...])

    grid_spec = pltpu.PrefetchScalarGridSpec(
        num_scalar_prefetch=1,
        grid=(NBLK,),
        in_specs=[
            pl.BlockSpec((BLK, 512), lambda i, s: (i, 0)),
            pl.BlockSpec((BLK, 1), lambda i, s: (i, 0)),
            pl.BlockSpec((G, 1), lambda i, s: (0, 0)),
            pl.BlockSpec((1024, 1024), lambda i, s: (0, 0)),
            pl.BlockSpec((1, 1024), lambda i, s: (0, 0)),
            pl.BlockSpec((1024, 128), lambda i, s: (0, 0)),
            pl.BlockSpec((1, 128), lambda i, s: (0, 0)),
        ],
        out_specs=pl.BlockSpec((G, 128), lambda i, s: (0, 0)),
        scratch_shapes=[
            pltpu.VMEM((G, 512), f32),
            pltpu.VMEM((G, 512), f32),
        ],
    )
    return pl.pallas_call(
        body,
        grid_spec=grid_spec,
        out_shape=jax.ShapeDtypeStruct((G, 128), f32),
    )(seg, h3, batchcol, counts2, Wf1, bf1, Wf2, bf2)


_SC_KERNELS = {}


def _deg_kernel(*args):
    if "deg" not in _SC_KERNELS:
        _SC_KERNELS["deg"] = _make_deg_kernel()
    return _SC_KERNELS["deg"](*args)


def _scatter(C, *args):
    if C not in _SC_KERNELS:
        _SC_KERNELS[C] = _make_scatter_kernel(C)
    return _SC_KERNELS[C](*args)


def _scatter1(*args):
    return _scatter(1, *args)


def _scatter2(*args):
    return _scatter(2, *args)


def _scatter4(*args):
    return _scatter(4, *args)


def kernel(x, edge_index, batch, W1, b1, W2, b2, W3, b3, Wf1, bf1, Wf2, bf2):
    src = edge_index[0]
    dst = edge_index[1]
    idx3 = jnp.stack([src.reshape(EB, 128), dst.reshape(EB, 128)], axis=1)
    zeros_n128 = jnp.zeros((N, 128), f32)
    ones_n128 = jnp.ones((128, 128), f32)
    batchcol = batch.reshape(N, 1)

    degp = _deg_kernel(dst, ones_n128, zeros_n128)
    y1, dinv, counts = _tck1(x, degp[0], degp[1], batchcol, W1)

    z1 = _scatter1(idx3, y1, zeros_n128)
    y20, y21 = _tck2(z1[0], z1[1], y1, dinv, b1.reshape(1, 128), W2)

    z2 = _scatter2(idx3, y20, y21, zeros_n128)
    y3c = _tck3(z2[0], z2[2], z2[1], z2[3], y20, y21, dinv,
                b2.reshape(1, 256), W3)

    z3 = _scatter4(idx3, *y3c, zeros_n128)
    h3 = _tck4([z3[c] for c in range(8)], list(y3c), dinv,
               b3.reshape(1, 512))

    seg = jnp.stack([batch[0::BLK], batch[BLK - 1::BLK]], axis=1).reshape(-1)
    counts2 = counts.reshape(G, 1)
    out = _tck5(seg, h3, batchcol, counts2, Wf1, bf1.reshape(1, 1024),
                Wf2, bf2.reshape(1, 128))
    return out


# speedup vs baseline: 1.2344x; 1.0339x over previous
import functools

import jax
import jax.numpy as jnp
from jax import lax
from jax.experimental import pallas as pl
from jax.experimental.pallas import tpu as pltpu
from jax.experimental.pallas import tpu_sc as plsc

N = 10000
E = 320000
G = 256

NCORE = 2
NSUB = 16
E_PER_CORE = E // NCORE
E_PER_SUB = E_PER_CORE // NSUB
NSTEP = E_PER_SUB // 128
REM = E_PER_SUB - NSTEP * 128
ROWS_MAIN = 624
ROWS_TAIL_OFF = ROWS_MAIN * NSUB
ROWS_TAIL = N - ROWS_TAIL_OFF

f32 = jnp.float32
i32 = jnp.int32


def _sc_mesh():
    return plsc.VectorSubcoreMesh(core_axis_name="c", subcore_axis_name="s",
                                  num_cores=NCORE, num_subcores=NSUB)


def _make_deg_kernel():
    @functools.partial(
        pl.kernel,
        out_type=jax.ShapeDtypeStruct((NCORE, N, 128), f32),
        mesh=_sc_mesh(),
        scratch_types=[
            pltpu.VMEM_SHARED((N, 128), f32),
            pltpu.VMEM((128, 128), f32),
            pltpu.VMEM((2, 128), i32),
            pltpu.VMEM((2, 128), i32),
            pltpu.SemaphoreType.DMA,
            pltpu.SemaphoreType.DMA,
        ],
    )
    def deg_kernel(idx_hbm, ones_hbm, zeros_hbm, degp_hbm, acc, ones_v,
                   idxa, idxb, semsa, semsb):
        c = lax.axis_index("c")
        s = lax.axis_index("s")
        r0 = s * ROWS_MAIN
        b0 = c * B_PER_CORE + 78 * s + jnp.minimum(s, 2)
        pltpu.sync_copy(ones_hbm, ones_v)
        pltpu.sync_copy(zeros_hbm.at[pl.ds(r0, ROWS_MAIN)],
                        acc.at[pl.ds(r0, ROWS_MAIN)])

        @pl.when(s == NSUB - 1)
        def _():
            pltpu.sync_copy(zeros_hbm.at[pl.ds(ROWS_TAIL_OFF, ROWS_TAIL)],
                            acc.at[pl.ds(ROWS_TAIL_OFF, ROWS_TAIL)])

        plsc.subcore_barrier()
        pltpu.sync_copy(idx_hbm.at[b0], idxa)

        def pair(p, carry):
            ba = b0 + 2 * p
            pltpu.async_copy(ones_v, acc.at[idxa.at[1]], semsa, add=True)

            @pl.when(p > 0)
            def _():
                pltpu.make_async_copy(ones_v, acc.at[idxb.at[1]],
                                      semsb).wait()

            pltpu.sync_copy(idx_hbm.at[ba + 1], idxb)
            pltpu.async_copy(ones_v, acc.at[idxb.at[1]], semsb, add=True)

            @pl.when(p < NP - 1)
            def _():
                pltpu.make_async_copy(ones_v, acc.at[idxa.at[1]],
                                      semsa).wait()
                pltpu.sync_copy(idx_hbm.at[ba + 2], idxa)
            return carry

        lax.fori_loop(0, NP, pair, 0)
        pltpu.make_async_copy(ones_v, acc.at[idxa.at[1]], semsa).wait()
        pltpu.make_async_copy(ones_v, acc.at[idxb.at[1]], semsb).wait()

        @pl.when(s < 2)
        def _():
            pltpu.sync_copy(idx_hbm.at[b0 + 2 * NP], idxa)
            pltpu.sync_copy(ones_v, acc.at[idxa.at[1]], add=True)

        plsc.subcore_barrier()
        pltpu.sync_copy(acc.at[pl.ds(r0, ROWS_MAIN)],
                        degp_hbm.at[c, pl.ds(r0, ROWS_MAIN)])

        @pl.when(s == NSUB - 1)
        def _():
            pltpu.sync_copy(acc.at[pl.ds(ROWS_TAIL_OFF, ROWS_TAIL)],
                            degp_hbm.at[c, pl.ds(ROWS_TAIL_OFF, ROWS_TAIL)])

    return deg_kernel


EB = E // 128
B_PER_CORE = EB // NCORE
NP = 39


def _make_scatter_kernel(C):
    scratch = [
        pltpu.VMEM_SHARED((N, 128), f32),
        pltpu.VMEM((2, 128), i32),
        pltpu.VMEM((2, 128), i32),
        pltpu.VMEM((128, 128), f32),
        pltpu.VMEM((128, 128), f32),
        pltpu.SemaphoreType.DMA,
        pltpu.SemaphoreType.DMA,
        pltpu.SemaphoreType.DMA,
        pltpu.SemaphoreType.DMA,
    ]

    @functools.partial(
        pl.kernel,
        out_type=jax.ShapeDtypeStruct((NCORE * C, N, 128), f32),
        mesh=_sc_mesh(),
        scratch_types=scratch,
    )
    def scatter_kernel(*refs):
        idx_hbm = refs[0]
        ys = refs[1:1 + C]
        zeros_hbm = refs[1 + C]
        z_hbm = refs[2 + C]
        (acc, idxa, idxb, rowsa, rowsb, sema, semb, semsa, semsb) = \
            refs[3 + C:]

        c = lax.axis_index("c")
        s = lax.axis_index("s")
        r0 = s * ROWS_MAIN
        b0 = c * B_PER_CORE + 78 * s + jnp.minimum(s, 2)

        for chunk in range(C):
            y = ys[chunk]
            pltpu.sync_copy(zeros_hbm.at[pl.ds(r0, ROWS_MAIN)],
                            acc.at[pl.ds(r0, ROWS_MAIN)])

            @pl.when(s == NSUB - 1)
            def _():
                pltpu.sync_copy(zeros_hbm.at[pl.ds(ROWS_TAIL_OFF, ROWS_TAIL)],
                                acc.at[pl.ds(ROWS_TAIL_OFF, ROWS_TAIL)])

            plsc.subcore_barrier()

            pltpu.sync_copy(idx_hbm.at[b0], idxa)
            pltpu.async_copy(y.at[idxa.at[0]], rowsa, sema)

            def pair(p, carry):
                ba = b0 + 2 * p
                bb = ba + 1

                @pl.when(p > 0)
                def _():
                    pltpu.make_async_copy(rowsb, acc.at[idxb.at[1]],
                                          semsb).wait()

                pltpu.sync_copy(idx_hbm.at[bb], idxb)
                pltpu.async_copy(y.at[idxb.at[0]], rowsb, semb)
                pltpu.make_async_copy(y.at[idxa.at[0]], rowsa, sema).wait()
                pltpu.async_copy(rowsa, acc.at[idxa.at[1]], semsa, add=True)

                @pl.when(p < NP - 1)
                def _():
                    pltpu.make_async_copy(rowsa, acc.at[idxa.at[1]],
                                          semsa).wait()
                    pltpu.sync_copy(idx_hbm.at[ba + 2], idxa)
                    pltpu.async_copy(y.at[idxa.at[0]], rowsa, sema)

                pltpu.make_async_copy(y.at[idxb.at[0]], rowsb, semb).wait()
                pltpu.async_copy(rowsb, acc.at[idxb.at[1]], semsb, add=True)
                return carry

            lax.fori_loop(0, NP, pair, 0)
            pltpu.make_async_copy(rowsa, acc.at[idxa.at[1]], semsa).wait()
            pltpu.make_async_copy(rowsb, acc.at[idxb.at[1]], semsb).wait()

            @pl.when(s < 2)
            def _():
                pltpu.sync_copy(idx_hbm.at[b0 + 2 * NP], idxa)
                pltpu.async_copy(y.at[idxa.at[0]], rowsa, sema).wait()
                pltpu.sync_copy(rowsa, acc.at[idxa.at[1]], add=True)

            plsc.subcore_barrier()
            zi = c * C + chunk
            pltpu.sync_copy(acc.at[pl.ds(r0, ROWS_MAIN)],
                            z_hbm.at[zi, pl.ds(r0, ROWS_MAIN)])

            @pl.when(s == NSUB - 1)
            def _():
                pltpu.sync_copy(acc.at[pl.ds(ROWS_TAIL_OFF, ROWS_TAIL)],
                                z_hbm.at[zi, pl.ds(ROWS_TAIL_OFF, ROWS_TAIL)])

            plsc.subcore_barrier()

    return scatter_kernel


BLK = 400
NBLK = N // BLK


def _tck1(x, dega, degb, batchcol, W1):
    def body(x_ref, da_ref, db_ref, b_ref, w_ref, y_ref, dinv_ref, cnt_ref):
        i = pl.program_id(0)
        deg = da_ref[:, 0:1] + db_ref[:, 0:1] + 1.0
        dinv = lax.rsqrt(deg)
        dinv_ref[...] = dinv
        y_ref[...] = dinv * jnp.dot(x_ref[...], w_ref[...],
                                    preferred_element_type=f32)
        oh = (b_ref[...] == lax.broadcasted_iota(i32, (1, G), 1)).astype(f32)
        csum = jnp.sum(oh, axis=0, keepdims=True)

        @pl.when(i == 0)
        def _():
            cnt_ref[...] = csum

        @pl.when(i > 0)
        def _():
            cnt_ref[...] += csum

    return pl.pallas_call(
        body,
        grid=(NBLK,),
        in_specs=[
            pl.BlockSpec((BLK, 128), lambda i: (i, 0)),
            pl.BlockSpec((BLK, 128), lambda i: (i, 0)),
            pl.BlockSpec((BLK, 128), lambda i: (i, 0)),
            pl.BlockSpec((BLK, 1), lambda i: (i, 0)),
            pl.BlockSpec((128, 128), lambda i: (0, 0)),
        ],
        out_specs=[
            pl.BlockSpec((BLK, 128), lambda i: (i, 0)),
            pl.BlockSpec((BLK, 1), lambda i: (i, 0)),
            pl.BlockSpec((1, G), lambda i: (0, 0)),
        ],
        out_shape=[
            jax.ShapeDtypeStruct((N, 128), f32),
            jax.ShapeDtypeStruct((N, 1), f32),
            jax.ShapeDtypeStruct((1, G), f32),
        ],
    )(x, dega, degb, batchcol, W1)


def _tck2(z1a, z1b, y1, dinv, b1, W2):
    def body(za_ref, zb_ref, y_ref, d_ref, b_ref, w_ref, o0_ref, o1_ref):
        d = d_ref[...]
        h = jnp.maximum(d * (za_ref[...] + zb_ref[...] + y_ref[...])
                        + b_ref[...], 0.0)
        y2 = d * jnp.dot(h, w_ref[...], preferred_element_type=f32)
        o0_ref[...] = y2[:, :128]
        o1_ref[...] = y2[:, 128:]

    return pl.pallas_call(
        body,
        grid=(NBLK,),
        in_specs=[
            pl.BlockSpec((BLK, 128), lambda i: (i, 0)),
            pl.BlockSpec((BLK, 128), lambda i: (i, 0)),
            pl.BlockSpec((BLK, 128), lambda i: (i, 0)),
            pl.BlockSpec((BLK, 1), lambda i: (i, 0)),
            pl.BlockSpec((1, 128), lambda i: (0, 0)),
            pl.BlockSpec((128, 256), lambda i: (0, 0)),
        ],
        out_specs=[
            pl.BlockSpec((BLK, 128), lambda i: (i, 0)),
            pl.BlockSpec((BLK, 128), lambda i: (i, 0)),
        ],
        out_shape=[
            jax.ShapeDtypeStruct((N, 128), f32),
            jax.ShapeDtypeStruct((N, 128), f32),
        ],
    )(z1a, z1b, y1, dinv, b1, W2)


def _tck3(za0, zb0, za1, zb1, y20, y21, dinv, b2, W3):
    def body(za0_ref, zb0_ref, za1_ref, zb1_ref, y0_ref, y1_ref, d_ref,
             b_ref, w_ref, o0_ref, o1_ref, o2_ref, o3_ref):
        d = d_ref[...]
        z = jnp.concatenate(
            [za0_ref[...] + zb0_ref[...] + y0_ref[...],
             za1_ref[...] + zb1_ref[...] + y1_ref[...]], axis=1)
        h = jnp.maximum(d * z + b_ref[...], 0.0)
        y3 = d * jnp.dot(h, w_ref[...], preferred_element_type=f32)
        o0_ref[...] = y3[:, 0:128]
        o1_ref[...] = y3[:, 128:256]
        o2_ref[...] = y3[:, 256:384]
        o3_ref[...] = y3[:, 384:512]

    blk = lambda w: pl.BlockSpec((BLK, w), lambda i: (i, 0))
    return pl.pallas_call(
        body,
        grid=(NBLK,),
        in_specs=[
            blk(128), blk(128), blk(128), blk(128), blk(128), blk(128),
            blk(1),
            pl.BlockSpec((1, 256), lambda i: (0, 0)),
            pl.BlockSpec((256, 512), lambda i: (0, 0)),
        ],
        out_specs=[blk(128), blk(128), blk(128), blk(128)],
        out_shape=[jax.ShapeDtypeStruct((N, 128), f32)] * 4,
    )(za0, zb0, za1, zb1, y20, y21, dinv, b2, W3)


def _tck4(zs, ys, dinv, b3):
    def body(*refs):
        za = refs[0:4]
        zb = refs[4:8]
        y = refs[8:12]
        d_ref, b_ref, h_ref = refs[12], refs[13], refs[14]
        d = d_ref[...]
        z = jnp.concatenate(
            [za[c][...] + zb[c][...] + y[c][...] for c in range(4)], axis=1)
        h_ref[...] = jnp.maximum(d * z + b_ref[...], 0.0)

    blk = lambda w: pl.BlockSpec((BLK, w), lambda i: (i, 0))
    return pl.pallas_call(
        body,
        grid=(NBLK,),
        in_specs=[blk(128)] * 12 + [
            blk(1),
            pl.BlockSpec((1, 512), lambda i: (0, 0)),
        ],
        out_specs=pl.BlockSpec((BLK, 512), lambda i: (i, 0)),
        out_shape=jax.ShapeDtypeStruct((N, 512), f32),
    )(*zs[0:4], *zs[4:8], *ys, dinv, b3)


def _tck5(seg, h3, batchcol, counts2, Wf1, bf1, Wf2, bf2):
    def body(seg_ref, h_ref, b_ref, cnt_ref, wf1_ref, bf1_ref, wf2_ref,
             bf2_ref, out_ref, mx_ref, sum_ref):
        i = pl.program_id(0)

        @pl.when(i == 0)
        def _():
            mx_ref[...] = jnp.full((G, 512), -1e30, f32)
            sum_ref[...] = jnp.zeros((G, 512), f32)

        h = h_ref[...]
        b = b_ref[...]
        oh = (b == lax.broadcasted_iota(i32, (1, G), 1)).astype(f32)
        sum_ref[...] += lax.dot_general(oh, h, (((0,), (0,)), ((), ())),
                                        preferred_element_type=f32)
        lo = seg_ref[2 * i]
        hi = seg_ref[2 * i + 1]

        def mbody(g, carry):
            m = b == g
            cand = jnp.max(jnp.where(m, h, -1e30), axis=0, keepdims=True)
            cur = mx_ref[pl.ds(g, 1), :]
            mx_ref[pl.ds(g, 1), :] = jnp.maximum(cur, cand)
            return carry

        lax.fori_loop(lo, hi + 1, mbody, 0)

        @pl.when(i == NBLK - 1)
        def _():
            cnt = cnt_ref[...]
            mean = sum_ref[...] / jnp.maximum(cnt, 1.0)
            mx = jnp.where(cnt > 0, mx_ref[...], 0.0)
            pooled = jnp.concatenate([mx, mean], axis=1)
            zf = jnp.maximum(
                jnp.dot(pooled, wf1_ref[...], preferred_element_type=f32)
                + bf1_ref[...], 0.0)
            out_ref[...] = (jnp.dot(zf, wf2_ref[...],
                                    preferred_element_type=f32) + bf2_ref[...])

    grid_spec = pltpu.PrefetchScalarGridSpec(
        num_scalar_prefetch=1,
        grid=(NBLK,),
        in_specs=[
            pl.BlockSpec((BLK, 512), lambda i, s: (i, 0)),
            pl.BlockSpec((BLK, 1), lambda i, s: (i, 0)),
            pl.BlockSpec((G, 1), lambda i, s: (0, 0)),
            pl.BlockSpec((1024, 1024), lambda i, s: (0, 0)),
            pl.BlockSpec((1, 1024), lambda i, s: (0, 0)),
            pl.BlockSpec((1024, 128), lambda i, s: (0, 0)),
            pl.BlockSpec((1, 128), lambda i, s: (0, 0)),
        ],
        out_specs=pl.BlockSpec((G, 128), lambda i, s: (0, 0)),
        scratch_shapes=[
            pltpu.VMEM((G, 512), f32),
            pltpu.VMEM((G, 512), f32),
        ],
    )
    return pl.pallas_call(
        body,
        grid_spec=grid_spec,
        out_shape=jax.ShapeDtypeStruct((G, 128), f32),
    )(seg, h3, batchcol, counts2, Wf1, bf1, Wf2, bf2)


_SC_KERNELS = {}


def _deg_kernel(*args):
    if "deg" not in _SC_KERNELS:
        _SC_KERNELS["deg"] = _make_deg_kernel()
    return _SC_KERNELS["deg"](*args)


def _scatter(C, *args):
    if C not in _SC_KERNELS:
        _SC_KERNELS[C] = _make_scatter_kernel(C)
    return _SC_KERNELS[C](*args)


def _scatter1(*args):
    return _scatter(1, *args)


def _scatter2(*args):
    return _scatter(2, *args)


def _scatter4(*args):
    return _scatter(4, *args)


def kernel(x, edge_index, batch, W1, b1, W2, b2, W3, b3, Wf1, bf1, Wf2, bf2):
    src = edge_index[0]
    dst = edge_index[1]
    idx3 = jnp.stack([src.reshape(EB, 128), dst.reshape(EB, 128)], axis=1)
    zeros_n128 = jnp.zeros((N, 128), f32)
    ones_n128 = jnp.ones((128, 128), f32)
    batchcol = batch.reshape(N, 1)

    degp = _deg_kernel(idx3, ones_n128, zeros_n128)
    y1, dinv, counts = _tck1(x, degp[0], degp[1], batchcol, W1)

    z1 = _scatter1(idx3, y1, zeros_n128)
    y20, y21 = _tck2(z1[0], z1[1], y1, dinv, b1.reshape(1, 128), W2)

    z2 = _scatter2(idx3, y20, y21, zeros_n128)
    y3c = _tck3(z2[0], z2[2], z2[1], z2[3], y20, y21, dinv,
                b2.reshape(1, 256), W3)

    z3 = _scatter4(idx3, *y3c, zeros_n128)
    h3 = _tck4([z3[c] for c in range(8)], list(y3c), dinv,
               b3.reshape(1, 512))

    seg = jnp.stack([batch[0::BLK], batch[BLK - 1::BLK]], axis=1).reshape(-1)
    counts2 = counts.reshape(G, 1)
    out = _tck5(seg, h3, batchcol, counts2, Wf1, bf1.reshape(1, 1024),
                Wf2, bf2.reshape(1, 128))
    return out


# speedup vs baseline: 1.2621x; 1.0224x over previous
import functools

import jax
import jax.numpy as jnp
from jax import lax
from jax.experimental import pallas as pl
from jax.experimental.pallas import tpu as pltpu
from jax.experimental.pallas import tpu_sc as plsc

N = 10000
E = 320000
G = 256

NCORE = 2
NSUB = 16
E_PER_CORE = E // NCORE
E_PER_SUB = E_PER_CORE // NSUB
NSTEP = E_PER_SUB // 128
REM = E_PER_SUB - NSTEP * 128
ROWS_MAIN = 624
ROWS_TAIL_OFF = ROWS_MAIN * NSUB
ROWS_TAIL = N - ROWS_TAIL_OFF

f32 = jnp.float32
i32 = jnp.int32


def _sc_mesh():
    return plsc.VectorSubcoreMesh(core_axis_name="c", subcore_axis_name="s",
                                  num_cores=NCORE, num_subcores=NSUB)


def _make_deg_kernel():
    @functools.partial(
        pl.kernel,
        out_type=jax.ShapeDtypeStruct((NCORE, N, 128), f32),
        mesh=_sc_mesh(),
        scratch_types=[
            pltpu.VMEM_SHARED((N, 128), f32),
            pltpu.VMEM((128, 128), f32),
            pltpu.VMEM((2, 128), i32),
            pltpu.VMEM((2, 128), i32),
            pltpu.SemaphoreType.DMA,
            pltpu.SemaphoreType.DMA,
        ],
    )
    def deg_kernel(idx_hbm, ones_hbm, zeros_hbm, degp_hbm, acc, ones_v,
                   idxa, idxb, semsa, semsb):
        c = lax.axis_index("c")
        s = lax.axis_index("s")
        r0 = s * ROWS_MAIN
        b0 = c * B_PER_CORE + 78 * s + jnp.minimum(s, 2)
        pltpu.sync_copy(ones_hbm, ones_v)
        pltpu.sync_copy(zeros_hbm.at[pl.ds(r0, ROWS_MAIN)],
                        acc.at[pl.ds(r0, ROWS_MAIN)])

        @pl.when(s == NSUB - 1)
        def _():
            pltpu.sync_copy(zeros_hbm.at[pl.ds(ROWS_TAIL_OFF, ROWS_TAIL)],
                            acc.at[pl.ds(ROWS_TAIL_OFF, ROWS_TAIL)])

        plsc.subcore_barrier()
        pltpu.sync_copy(idx_hbm.at[b0], idxa)

        def pair(p, carry):
            ba = b0 + 2 * p
            pltpu.async_copy(ones_v, acc.at[idxa.at[1]], semsa, add=True)

            @pl.when(p > 0)
            def _():
                pltpu.make_async_copy(ones_v, acc.at[idxb.at[1]],
                                      semsb).wait()

            pltpu.sync_copy(idx_hbm.at[ba + 1], idxb)
            pltpu.async_copy(ones_v, acc.at[idxb.at[1]], semsb, add=True)

            @pl.when(p < NP - 1)
            def _():
                pltpu.make_async_copy(ones_v, acc.at[idxa.at[1]],
                                      semsa).wait()
                pltpu.sync_copy(idx_hbm.at[ba + 2], idxa)
            return carry

        lax.fori_loop(0, NP, pair, 0)
        pltpu.make_async_copy(ones_v, acc.at[idxa.at[1]], semsa).wait()
        pltpu.make_async_copy(ones_v, acc.at[idxb.at[1]], semsb).wait()

        @pl.when(s < 2)
        def _():
            pltpu.sync_copy(idx_hbm.at[b0 + 2 * NP], idxa)
            pltpu.sync_copy(ones_v, acc.at[idxa.at[1]], add=True)

        plsc.subcore_barrier()
        pltpu.sync_copy(acc.at[pl.ds(r0, ROWS_MAIN)],
                        degp_hbm.at[c, pl.ds(r0, ROWS_MAIN)])

        @pl.when(s == NSUB - 1)
        def _():
            pltpu.sync_copy(acc.at[pl.ds(ROWS_TAIL_OFF, ROWS_TAIL)],
                            degp_hbm.at[c, pl.ds(ROWS_TAIL_OFF, ROWS_TAIL)])

    return deg_kernel


EB = E // 128
B_PER_CORE = EB // NCORE
NP = 39


def _make_scatter_kernel(C):
    scratch = [
        pltpu.VMEM_SHARED((N, 128), f32),
        pltpu.VMEM((2, 128), i32),
        pltpu.VMEM((2, 128), i32),
        pltpu.VMEM((128, 128), f32),
        pltpu.VMEM((128, 128), f32),
        pltpu.SemaphoreType.DMA,
        pltpu.SemaphoreType.DMA,
        pltpu.SemaphoreType.DMA,
        pltpu.SemaphoreType.DMA,
    ]

    @functools.partial(
        pl.kernel,
        out_type=jax.ShapeDtypeStruct((NCORE * C, N, 128), f32),
        mesh=_sc_mesh(),
        scratch_types=scratch,
    )
    def scatter_kernel(*refs):
        idx_hbm = refs[0]
        ys = refs[1:1 + C]
        zeros_hbm = refs[1 + C]
        z_hbm = refs[2 + C]
        (acc, idxa, idxb, rowsa, rowsb, sema, semb, semsa, semsb) = \
            refs[3 + C:]

        c = lax.axis_index("c")
        s = lax.axis_index("s")
        r0 = s * ROWS_MAIN
        b0 = c * B_PER_CORE + 78 * s + jnp.minimum(s, 2)

        for chunk in range(C):
            y = ys[chunk]
            pltpu.sync_copy(zeros_hbm.at[pl.ds(r0, ROWS_MAIN)],
                            acc.at[pl.ds(r0, ROWS_MAIN)])

            @pl.when(s == NSUB - 1)
            def _():
                pltpu.sync_copy(zeros_hbm.at[pl.ds(ROWS_TAIL_OFF, ROWS_TAIL)],
                                acc.at[pl.ds(ROWS_TAIL_OFF, ROWS_TAIL)])

            plsc.subcore_barrier()

            pltpu.sync_copy(idx_hbm.at[b0], idxa)
            pltpu.async_copy(y.at[idxa.at[0]], rowsa, sema)

            def pair(p, carry):
                ba = b0 + 2 * p
                bb = ba + 1

                @pl.when(p > 0)
                def _():
                    pltpu.make_async_copy(rowsb, acc.at[idxb.at[1]],
                                          semsb).wait()

                pltpu.sync_copy(idx_hbm.at[bb], idxb)
                pltpu.async_copy(y.at[idxb.at[0]], rowsb, semb)
                pltpu.make_async_copy(y.at[idxa.at[0]], rowsa, sema).wait()
                pltpu.async_copy(rowsa, acc.at[idxa.at[1]], semsa, add=True)

                @pl.when(p < NP - 1)
                def _():
                    pltpu.make_async_copy(rowsa, acc.at[idxa.at[1]],
                                          semsa).wait()
                    pltpu.sync_copy(idx_hbm.at[ba + 2], idxa)
                    pltpu.async_copy(y.at[idxa.at[0]], rowsa, sema)

                pltpu.make_async_copy(y.at[idxb.at[0]], rowsb, semb).wait()
                pltpu.async_copy(rowsb, acc.at[idxb.at[1]], semsb, add=True)
                return carry

            lax.fori_loop(0, NP, pair, 0)
            pltpu.make_async_copy(rowsa, acc.at[idxa.at[1]], semsa).wait()
            pltpu.make_async_copy(rowsb, acc.at[idxb.at[1]], semsb).wait()

            @pl.when(s < 2)
            def _():
                pltpu.sync_copy(idx_hbm.at[b0 + 2 * NP], idxa)
                pltpu.async_copy(y.at[idxa.at[0]], rowsa, sema).wait()
                pltpu.sync_copy(rowsa, acc.at[idxa.at[1]], add=True)

            plsc.subcore_barrier()
            zi = c * C + chunk
            pltpu.sync_copy(acc.at[pl.ds(r0, ROWS_MAIN)],
                            z_hbm.at[zi, pl.ds(r0, ROWS_MAIN)])

            @pl.when(s == NSUB - 1)
            def _():
                pltpu.sync_copy(acc.at[pl.ds(ROWS_TAIL_OFF, ROWS_TAIL)],
                                z_hbm.at[zi, pl.ds(ROWS_TAIL_OFF, ROWS_TAIL)])

            plsc.subcore_barrier()

    return scatter_kernel


BLK = 400
NBLK = N // BLK


def _tck1(x, dega, degb, batchcol, W1):
    def body(x_ref, da_ref, db_ref, b_ref, w_ref, y_ref, dinv_ref, cnt_ref):
        i = pl.program_id(0)
        deg = da_ref[:, 0:1] + db_ref[:, 0:1] + 1.0
        dinv = lax.rsqrt(deg)
        dinv_ref[...] = dinv
        y_ref[...] = dinv * jnp.dot(x_ref[...], w_ref[...],
                                    preferred_element_type=f32)
        oh = (b_ref[...] == lax.broadcasted_iota(i32, (1, G), 1)).astype(f32)
        csum = jnp.sum(oh, axis=0, keepdims=True)

        @pl.when(i == 0)
        def _():
            cnt_ref[...] = csum

        @pl.when(i > 0)
        def _():
            cnt_ref[...] += csum

    return pl.pallas_call(
        body,
        grid=(NBLK,),
        in_specs=[
            pl.BlockSpec((BLK, 128), lambda i: (i, 0)),
            pl.BlockSpec((BLK, 128), lambda i: (i, 0)),
            pl.BlockSpec((BLK, 128), lambda i: (i, 0)),
            pl.BlockSpec((BLK, 1), lambda i: (i, 0)),
            pl.BlockSpec((128, 128), lambda i: (0, 0)),
        ],
        out_specs=[
            pl.BlockSpec((BLK, 128), lambda i: (i, 0)),
            pl.BlockSpec((BLK, 1), lambda i: (i, 0)),
            pl.BlockSpec((1, G), lambda i: (0, 0)),
        ],
        out_shape=[
            jax.ShapeDtypeStruct((N, 128), f32),
            jax.ShapeDtypeStruct((N, 1), f32),
            jax.ShapeDtypeStruct((1, G), f32),
        ],
    )(x, dega, degb, batchcol, W1)


def _tck2(z1a, z1b, y1, dinv, b1, W2):
    def body(za_ref, zb_ref, y_ref, d_ref, b_ref, w_ref, o0_ref, o1_ref):
        d = d_ref[...]
        h = jnp.maximum(d * (za_ref[...] + zb_ref[...] + y_ref[...])
                        + b_ref[...], 0.0)
        y2 = d * jnp.dot(h, w_ref[...], preferred_element_type=f32)
        o0_ref[...] = y2[:, :128]
        o1_ref[...] = y2[:, 128:]

    return pl.pallas_call(
        body,
        grid=(NBLK,),
        in_specs=[
            pl.BlockSpec((BLK, 128), lambda i: (i, 0)),
            pl.BlockSpec((BLK, 128), lambda i: (i, 0)),
            pl.BlockSpec((BLK, 128), lambda i: (i, 0)),
            pl.BlockSpec((BLK, 1), lambda i: (i, 0)),
            pl.BlockSpec((1, 128), lambda i: (0, 0)),
            pl.BlockSpec((128, 256), lambda i: (0, 0)),
        ],
        out_specs=[
            pl.BlockSpec((BLK, 128), lambda i: (i, 0)),
            pl.BlockSpec((BLK, 128), lambda i: (i, 0)),
        ],
        out_shape=[
            jax.ShapeDtypeStruct((N, 128), f32),
            jax.ShapeDtypeStruct((N, 128), f32),
        ],
    )(z1a, z1b, y1, dinv, b1, W2)


def _tck3(za0, zb0, za1, zb1, y20, y21, dinv, b2, W3):
    def body(za0_ref, zb0_ref, za1_ref, zb1_ref, y0_ref, y1_ref, d_ref,
             b_ref, w_ref, o0_ref, o1_ref, o2_ref, o3_ref):
        d = d_ref[...]
        z = jnp.concatenate(
            [za0_ref[...] + zb0_ref[...] + y0_ref[...],
             za1_ref[...] + zb1_ref[...] + y1_ref[...]], axis=1)
        h = jnp.maximum(d * z + b_ref[...], 0.0)
        y3 = d * jnp.dot(h, w_ref[...], preferred_element_type=f32)
        o0_ref[...] = y3[:, 0:128]
        o1_ref[...] = y3[:, 128:256]
        o2_ref[...] = y3[:, 256:384]
        o3_ref[...] = y3[:, 384:512]

    blk = lambda w: pl.BlockSpec((BLK, w), lambda i: (i, 0))
    return pl.pallas_call(
        body,
        grid=(NBLK,),
        in_specs=[
            blk(128), blk(128), blk(128), blk(128), blk(128), blk(128),
            blk(1),
            pl.BlockSpec((1, 256), lambda i: (0, 0)),
            pl.BlockSpec((256, 512), lambda i: (0, 0)),
        ],
        out_specs=[blk(128), blk(128), blk(128), blk(128)],
        out_shape=[jax.ShapeDtypeStruct((N, 128), f32)] * 4,
    )(za0, zb0, za1, zb1, y20, y21, dinv, b2, W3)


def _tck45(seg, zs, ys, dinv, b3, batchcol, counts2, Wf1, bf1, Wf2, bf2):
    def body(seg_ref, *refs):
        za = refs[0:4]
        zb = refs[4:8]
        y = refs[8:12]
        (d_ref, b3_ref, b_ref, cnt_ref, wf1_ref, bf1_ref, wf2_ref, bf2_ref,
         out_ref, mx_ref, sum_ref) = refs[12:]
        i = pl.program_id(0)

        @pl.when(i == 0)
        def _():
            mx_ref[...] = jnp.full((G, 512), -1e30, f32)
            sum_ref[...] = jnp.zeros((G, 512), f32)

        z = jnp.concatenate(
            [za[c][...] + zb[c][...] + y[c][...] for c in range(4)], axis=1)
        h = jnp.maximum(d_ref[...] * z + b3_ref[...], 0.0)
        b = b_ref[...]
        oh = (b == lax.broadcasted_iota(i32, (1, G), 1)).astype(f32)
        sum_ref[...] += lax.dot_general(oh, h, (((0,), (0,)), ((), ())),
                                        preferred_element_type=f32)
        lo = seg_ref[2 * i]
        hi = seg_ref[2 * i + 1]

        def mbody(g, carry):
            m = b == g
            cand = jnp.max(jnp.where(m, h, -1e30), axis=0, keepdims=True)
            cur = mx_ref[pl.ds(g, 1), :]
            mx_ref[pl.ds(g, 1), :] = jnp.maximum(cur, cand)
            return carry

        lax.fori_loop(lo, hi + 1, mbody, 0)

        @pl.when(i == NBLK - 1)
        def _():
            cnt = cnt_ref[...]
            mean = sum_ref[...] / jnp.maximum(cnt, 1.0)
            mx = jnp.where(cnt > 0, mx_ref[...], 0.0)
            pooled = jnp.concatenate([mx, mean], axis=1)
            zf = jnp.maximum(
                jnp.dot(pooled, wf1_ref[...], preferred_element_type=f32)
                + bf1_ref[...], 0.0)
            out_ref[...] = (jnp.dot(zf, wf2_ref[...],
                                    preferred_element_type=f32) + bf2_ref[...])

    blk = lambda w: pl.BlockSpec((BLK, w), lambda i, s: (i, 0))
    grid_spec = pltpu.PrefetchScalarGridSpec(
        num_scalar_prefetch=1,
        grid=(NBLK,),
        in_specs=[blk(128)] * 12 + [
            blk(1),
            pl.BlockSpec((1, 512), lambda i, s: (0, 0)),
            blk(1),
            pl.BlockSpec((G, 1), lambda i, s: (0, 0)),
            pl.BlockSpec((1024, 1024), lambda i, s: (0, 0)),
            pl.BlockSpec((1, 1024), lambda i, s: (0, 0)),
            pl.BlockSpec((1024, 128), lambda i, s: (0, 0)),
            pl.BlockSpec((1, 128), lambda i, s: (0, 0)),
        ],
        out_specs=pl.BlockSpec((G, 128), lambda i, s: (0, 0)),
        scratch_shapes=[
            pltpu.VMEM((G, 512), f32),
            pltpu.VMEM((G, 512), f32),
        ],
    )
    return pl.pallas_call(
        body,
        grid_spec=grid_spec,
        out_shape=jax.ShapeDtypeStruct((G, 128), f32),
    )(seg, *zs[0:4], *zs[4:8], *ys, dinv, b3, batchcol, counts2,
      Wf1, bf1, Wf2, bf2)


_SC_KERNELS = {}


def _deg_kernel(*args):
    if "deg" not in _SC_KERNELS:
        _SC_KERNELS["deg"] = _make_deg_kernel()
    return _SC_KERNELS["deg"](*args)


def _scatter(C, *args):
    if C not in _SC_KERNELS:
        _SC_KERNELS[C] = _make_scatter_kernel(C)
    return _SC_KERNELS[C](*args)


def _scatter1(*args):
    return _scatter(1, *args)


def _scatter2(*args):
    return _scatter(2, *args)


def _scatter4(*args):
    return _scatter(4, *args)


def kernel(x, edge_index, batch, W1, b1, W2, b2, W3, b3, Wf1, bf1, Wf2, bf2):
    src = edge_index[0]
    dst = edge_index[1]
    idx3 = jnp.stack([src.reshape(EB, 128), dst.reshape(EB, 128)], axis=1)
    zeros_n128 = jnp.zeros((N, 128), f32)
    ones_n128 = jnp.ones((128, 128), f32)
    batchcol = batch.reshape(N, 1)

    degp = _deg_kernel(idx3, ones_n128, zeros_n128)
    y1, dinv, counts = _tck1(x, degp[0], degp[1], batchcol, W1)

    z1 = _scatter1(idx3, y1, zeros_n128)
    y20, y21 = _tck2(z1[0], z1[1], y1, dinv, b1.reshape(1, 128), W2)

    z2 = _scatter2(idx3, y20, y21, zeros_n128)
    y3c = _tck3(z2[0], z2[2], z2[1], z2[3], y20, y21, dinv,
                b2.reshape(1, 256), W3)

    z3 = _scatter4(idx3, *y3c, zeros_n128)
    seg = jnp.stack([batch[0::BLK], batch[BLK - 1::BLK]], axis=1).reshape(-1)
    counts2 = counts.reshape(G, 1)
    out = _tck45(seg, [z3[c] for c in range(8)], list(y3c), dinv,
                 b3.reshape(1, 512), batchcol, counts2,
                 Wf1, bf1.reshape(1, 1024), Wf2, bf2.reshape(1, 128))
    return out


# speedup vs baseline: 1.2656x; 1.0028x over previous
import functools

import jax
import jax.numpy as jnp
from jax import lax
from jax.experimental import pallas as pl
from jax.experimental.pallas import tpu as pltpu
from jax.experimental.pallas import tpu_sc as plsc

N = 10000
E = 320000
G = 256

NCORE = 2
NSUB = 16
E_PER_CORE = E // NCORE
E_PER_SUB = E_PER_CORE // NSUB
NSTEP = E_PER_SUB // 128
REM = E_PER_SUB - NSTEP * 128
ROWS_MAIN = 624
ROWS_TAIL_OFF = ROWS_MAIN * NSUB
ROWS_TAIL = N - ROWS_TAIL_OFF

f32 = jnp.float32
i32 = jnp.int32


def _sc_mesh():
    return plsc.VectorSubcoreMesh(core_axis_name="c", subcore_axis_name="s",
                                  num_cores=NCORE, num_subcores=NSUB)


def _make_deg_kernel():
    @functools.partial(
        pl.kernel,
        out_type=jax.ShapeDtypeStruct((NCORE, N, 128), f32),
        mesh=_sc_mesh(),
        scratch_types=[
            pltpu.VMEM_SHARED((N, 128), f32),
            pltpu.VMEM((128, 128), f32),
            pltpu.VMEM((2, 128), i32),
            pltpu.VMEM((2, 128), i32),
            pltpu.SemaphoreType.DMA,
            pltpu.SemaphoreType.DMA,
        ],
    )
    def deg_kernel(idx_hbm, ones_hbm, zeros_hbm, degp_hbm, acc, ones_v,
                   idxa, idxb, semsa, semsb):
        c = lax.axis_index("c")
        s = lax.axis_index("s")
        r0 = s * ROWS_MAIN
        b0 = c * B_PER_CORE + 78 * s + jnp.minimum(s, 2)
        pltpu.sync_copy(ones_hbm, ones_v)
        pltpu.sync_copy(zeros_hbm.at[pl.ds(r0, ROWS_MAIN)],
                        acc.at[pl.ds(r0, ROWS_MAIN)])

        @pl.when(s == NSUB - 1)
        def _():
            pltpu.sync_copy(zeros_hbm.at[pl.ds(ROWS_TAIL_OFF, ROWS_TAIL)],
                            acc.at[pl.ds(ROWS_TAIL_OFF, ROWS_TAIL)])

        plsc.subcore_barrier()
        pltpu.sync_copy(idx_hbm.at[b0], idxa)

        def pair(p, carry):
            ba = b0 + 2 * p
            pltpu.async_copy(ones_v, acc.at[idxa.at[1]], semsa, add=True)

            @pl.when(p > 0)
            def _():
                pltpu.make_async_copy(ones_v, acc.at[idxb.at[1]],
                                      semsb).wait()

            pltpu.sync_copy(idx_hbm.at[ba + 1], idxb)
            pltpu.async_copy(ones_v, acc.at[idxb.at[1]], semsb, add=True)

            @pl.when(p < NP - 1)
            def _():
                pltpu.make_async_copy(ones_v, acc.at[idxa.at[1]],
                                      semsa).wait()
                pltpu.sync_copy(idx_hbm.at[ba + 2], idxa)
            return carry

        lax.fori_loop(0, NP, pair, 0)
        pltpu.make_async_copy(ones_v, acc.at[idxa.at[1]], semsa).wait()
        pltpu.make_async_copy(ones_v, acc.at[idxb.at[1]], semsb).wait()

        @pl.when(s < 2)
        def _():
            pltpu.sync_copy(idx_hbm.at[b0 + 2 * NP], idxa)
            pltpu.sync_copy(ones_v, acc.at[idxa.at[1]], add=True)

        plsc.subcore_barrier()
        pltpu.sync_copy(acc.at[pl.ds(r0, ROWS_MAIN)],
                        degp_hbm.at[c, pl.ds(r0, ROWS_MAIN)])

        @pl.when(s == NSUB - 1)
        def _():
            pltpu.sync_copy(acc.at[pl.ds(ROWS_TAIL_OFF, ROWS_TAIL)],
                            degp_hbm.at[c, pl.ds(ROWS_TAIL_OFF, ROWS_TAIL)])

    return deg_kernel


EB = E // 128
B_PER_CORE = EB // NCORE
NP = 39


def _make_scatter_kernel(C):
    scratch = [
        pltpu.VMEM_SHARED((N, 128), f32),
        pltpu.VMEM((2, 128), i32),
        pltpu.VMEM((2, 128), i32),
        pltpu.VMEM((128, 128), f32),
        pltpu.VMEM((128, 128), f32),
        pltpu.SemaphoreType.DMA,
        pltpu.SemaphoreType.DMA,
        pltpu.SemaphoreType.DMA,
        pltpu.SemaphoreType.DMA,
    ]

    @functools.partial(
        pl.kernel,
        out_type=jax.ShapeDtypeStruct((NCORE * C, N, 128), f32),
        mesh=_sc_mesh(),
        scratch_types=scratch,
    )
    def scatter_kernel(*refs):
        idx_hbm = refs[0]
        ys = refs[1:1 + C]
        zeros_hbm = refs[1 + C]
        z_hbm = refs[2 + C]
        (acc, idxa, idxb, rowsa, rowsb, sema, semb, semsa, semsb) = \
            refs[3 + C:]

        c = lax.axis_index("c")
        s = lax.axis_index("s")
        r0 = s * ROWS_MAIN
        b0 = c * B_PER_CORE + 78 * s + jnp.minimum(s, 2)

        for chunk in range(C):
            y = ys[chunk]
            pltpu.sync_copy(zeros_hbm.at[pl.ds(r0, ROWS_MAIN)],
                            acc.at[pl.ds(r0, ROWS_MAIN)])

            @pl.when(s == NSUB - 1)
            def _():
                pltpu.sync_copy(zeros_hbm.at[pl.ds(ROWS_TAIL_OFF, ROWS_TAIL)],
                                acc.at[pl.ds(ROWS_TAIL_OFF, ROWS_TAIL)])

            plsc.subcore_barrier()

            pltpu.sync_copy(idx_hbm.at[b0], idxa)
            pltpu.async_copy(y.at[idxa.at[0]], rowsa, sema)

            def pair(p, carry):
                ba = b0 + 2 * p
                bb = ba + 1

                @pl.when(p > 0)
                def _():
                    pltpu.make_async_copy(rowsb, acc.at[idxb.at[1]],
                                          semsb).wait()

                pltpu.sync_copy(idx_hbm.at[bb], idxb)
                pltpu.async_copy(y.at[idxb.at[0]], rowsb, semb)
                pltpu.make_async_copy(y.at[idxa.at[0]], rowsa, sema).wait()
                pltpu.async_copy(rowsa, acc.at[idxa.at[1]], semsa, add=True)

                @pl.when(p < NP - 1)
                def _():
                    pltpu.make_async_copy(rowsa, acc.at[idxa.at[1]],
                                          semsa).wait()
                    pltpu.sync_copy(idx_hbm.at[ba + 2], idxa)
                    pltpu.async_copy(y.at[idxa.at[0]], rowsa, sema)

                pltpu.make_async_copy(y.at[idxb.at[0]], rowsb, semb).wait()
                pltpu.async_copy(rowsb, acc.at[idxb.at[1]], semsb, add=True)
                return carry

            lax.fori_loop(0, NP, pair, 0)
            pltpu.make_async_copy(rowsa, acc.at[idxa.at[1]], semsa).wait()
            pltpu.make_async_copy(rowsb, acc.at[idxb.at[1]], semsb).wait()

            @pl.when(s < 2)
            def _():
                pltpu.sync_copy(idx_hbm.at[b0 + 2 * NP], idxa)
                pltpu.async_copy(y.at[idxa.at[0]], rowsa, sema).wait()
                pltpu.sync_copy(rowsa, acc.at[idxa.at[1]], add=True)

            plsc.subcore_barrier()
            zi = c * C + chunk
            pltpu.sync_copy(acc.at[pl.ds(r0, ROWS_MAIN)],
                            z_hbm.at[zi, pl.ds(r0, ROWS_MAIN)])

            @pl.when(s == NSUB - 1)
            def _():
                pltpu.sync_copy(acc.at[pl.ds(ROWS_TAIL_OFF, ROWS_TAIL)],
                                z_hbm.at[zi, pl.ds(ROWS_TAIL_OFF, ROWS_TAIL)])

            plsc.subcore_barrier()

    return scatter_kernel


BLK = 400
NBLK = N // BLK


def _tck1a(x, batchcol, W1):
    def body(x_ref, b_ref, w_ref, y_ref, cnt_ref):
        i = pl.program_id(0)
        y_ref[...] = jnp.dot(x_ref[...], w_ref[...],
                             preferred_element_type=f32)
        oh = (b_ref[...] == lax.broadcasted_iota(i32, (1, G), 1)).astype(f32)
        csum = jnp.sum(oh, axis=0, keepdims=True)

        @pl.when(i == 0)
        def _():
            cnt_ref[...] = csum

        @pl.when(i > 0)
        def _():
            cnt_ref[...] += csum

    return pl.pallas_call(
        body,
        grid=(NBLK,),
        in_specs=[
            pl.BlockSpec((BLK, 128), lambda i: (i, 0)),
            pl.BlockSpec((BLK, 1), lambda i: (i, 0)),
            pl.BlockSpec((128, 128), lambda i: (0, 0)),
        ],
        out_specs=[
            pl.BlockSpec((BLK, 128), lambda i: (i, 0)),
            pl.BlockSpec((1, G), lambda i: (0, 0)),
        ],
        out_shape=[
            jax.ShapeDtypeStruct((N, 128), f32),
            jax.ShapeDtypeStruct((1, G), f32),
        ],
    )(x, batchcol, W1)


def _tck1b(y1raw, dega, degb):
    def body(y_ref, da_ref, db_ref, out_ref, dinv_ref):
        deg = da_ref[:, 0:1] + db_ref[:, 0:1] + 1.0
        dinv = lax.rsqrt(deg)
        dinv_ref[...] = dinv
        out_ref[...] = dinv * y_ref[...]

    return pl.pallas_call(
        body,
        grid=(NBLK,),
        in_specs=[
            pl.BlockSpec((BLK, 128), lambda i: (i, 0)),
            pl.BlockSpec((BLK, 128), lambda i: (i, 0)),
            pl.BlockSpec((BLK, 128), lambda i: (i, 0)),
        ],
        out_specs=[
            pl.BlockSpec((BLK, 128), lambda i: (i, 0)),
            pl.BlockSpec((BLK, 1), lambda i: (i, 0)),
        ],
        out_shape=[
            jax.ShapeDtypeStruct((N, 128), f32),
            jax.ShapeDtypeStruct((N, 1), f32),
        ],
    )(y1raw, dega, degb)


def _tck2(z1a, z1b, y1, dinv, b1, W2):
    def body(za_ref, zb_ref, y_ref, d_ref, b_ref, w_ref, o0_ref, o1_ref):
        d = d_ref[...]
        h = jnp.maximum(d * (za_ref[...] + zb_ref[...] + y_ref[...])
                        + b_ref[...], 0.0)
        y2 = d * jnp.dot(h, w_ref[...], preferred_element_type=f32)
        o0_ref[...] = y2[:, :128]
        o1_ref[...] = y2[:, 128:]

    return pl.pallas_call(
        body,
        grid=(NBLK,),
        in_specs=[
            pl.BlockSpec((BLK, 128), lambda i: (i, 0)),
            pl.BlockSpec((BLK, 128), lambda i: (i, 0)),
            pl.BlockSpec((BLK, 128), lambda i: (i, 0)),
            pl.BlockSpec((BLK, 1), lambda i: (i, 0)),
            pl.BlockSpec((1, 128), lambda i: (0, 0)),
            pl.BlockSpec((128, 256), lambda i: (0, 0)),
        ],
        out_specs=[
            pl.BlockSpec((BLK, 128), lambda i: (i, 0)),
            pl.BlockSpec((BLK, 128), lambda i: (i, 0)),
        ],
        out_shape=[
            jax.ShapeDtypeStruct((N, 128), f32),
            jax.ShapeDtypeStruct((N, 128), f32),
        ],
    )(z1a, z1b, y1, dinv, b1, W2)


def _tck3(za0, zb0, za1, zb1, y20, y21, dinv, b2, W3):
    def body(za0_ref, zb0_ref, za1_ref, zb1_ref, y0_ref, y1_ref, d_ref,
             b_ref, w_ref, o0_ref, o1_ref, o2_ref, o3_ref):
        d = d_ref[...]
        z = jnp.concatenate(
            [za0_ref[...] + zb0_ref[...] + y0_ref[...],
             za1_ref[...] + zb1_ref[...] + y1_ref[...]], axis=1)
        h = jnp.maximum(d * z + b_ref[...], 0.0)
        y3 = d * jnp.dot(h, w_ref[...], preferred_element_type=f32)
        o0_ref[...] = y3[:, 0:128]
        o1_ref[...] = y3[:, 128:256]
        o2_ref[...] = y3[:, 256:384]
        o3_ref[...] = y3[:, 384:512]

    blk = lambda w: pl.BlockSpec((BLK, w), lambda i: (i, 0))
    return pl.pallas_call(
        body,
        grid=(NBLK,),
        in_specs=[
            blk(128), blk(128), blk(128), blk(128), blk(128), blk(128),
            blk(1),
            pl.BlockSpec((1, 256), lambda i: (0, 0)),
            pl.BlockSpec((256, 512), lambda i: (0, 0)),
        ],
        out_specs=[blk(128), blk(128), blk(128), blk(128)],
        out_shape=[jax.ShapeDtypeStruct((N, 128), f32)] * 4,
    )(za0, zb0, za1, zb1, y20, y21, dinv, b2, W3)


def _tck45(seg, zs, ys, dinv, b3, batchcol, counts2, Wf1, bf1, Wf2, bf2):
    def body(seg_ref, *refs):
        za = refs[0:4]
        zb = refs[4:8]
        y = refs[8:12]
        (d_ref, b3_ref, b_ref, cnt_ref, wf1_ref, bf1_ref, wf2_ref, bf2_ref,
         out_ref, mx_ref, sum_ref) = refs[12:]
        i = pl.program_id(0)

        @pl.when(i == 0)
        def _():
            mx_ref[...] = jnp.full((G, 512), -1e30, f32)
            sum_ref[...] = jnp.zeros((G, 512), f32)

        z = jnp.concatenate(
            [za[c][...] + zb[c][...] + y[c][...] for c in range(4)], axis=1)
        h = jnp.maximum(d_ref[...] * z + b3_ref[...], 0.0)
        b = b_ref[...]
        oh = (b == lax.broadcasted_iota(i32, (1, G), 1)).astype(f32)
        sum_ref[...] += lax.dot_general(oh, h, (((0,), (0,)), ((), ())),
                                        preferred_element_type=f32)
        lo = seg_ref[2 * i]
        hi = seg_ref[2 * i + 1]

        def mbody(g, carry):
            m = b == g
            cand = jnp.max(jnp.where(m, h, -1e30), axis=0, keepdims=True)
            cur = mx_ref[pl.ds(g, 1), :]
            mx_ref[pl.ds(g, 1), :] = jnp.maximum(cur, cand)
            return carry

        lax.fori_loop(lo, hi + 1, mbody, 0)

        @pl.when(i == NBLK - 1)
        def _():
            cnt = cnt_ref[...]
            mean = sum_ref[...] / jnp.maximum(cnt, 1.0)
            mx = jnp.where(cnt > 0, mx_ref[...], 0.0)
            pooled = jnp.concatenate([mx, mean], axis=1)
            zf = jnp.maximum(
                jnp.dot(pooled, wf1_ref[...], preferred_element_type=f32)
                + bf1_ref[...], 0.0)
            out_ref[...] = (jnp.dot(zf, wf2_ref[...],
                                    preferred_element_type=f32) + bf2_ref[...])

    blk = lambda w: pl.BlockSpec((BLK, w), lambda i, s: (i, 0))
    grid_spec = pltpu.PrefetchScalarGridSpec(
        num_scalar_prefetch=1,
        grid=(NBLK,),
        in_specs=[blk(128)] * 12 + [
            blk(1),
            pl.BlockSpec((1, 512), lambda i, s: (0, 0)),
            blk(1),
            pl.BlockSpec((G, 1), lambda i, s: (0, 0)),
            pl.BlockSpec((1024, 1024), lambda i, s: (0, 0)),
            pl.BlockSpec((1, 1024), lambda i, s: (0, 0)),
            pl.BlockSpec((1024, 128), lambda i, s: (0, 0)),
            pl.BlockSpec((1, 128), lambda i, s: (0, 0)),
        ],
        out_specs=pl.BlockSpec((G, 128), lambda i, s: (0, 0)),
        scratch_shapes=[
            pltpu.VMEM((G, 512), f32),
            pltpu.VMEM((G, 512), f32),
        ],
    )
    return pl.pallas_call(
        body,
        grid_spec=grid_spec,
        out_shape=jax.ShapeDtypeStruct((G, 128), f32),
    )(seg, *zs[0:4], *zs[4:8], *ys, dinv, b3, batchcol, counts2,
      Wf1, bf1, Wf2, bf2)


_SC_KERNELS = {}


def _deg_kernel(*args):
    if "deg" not in _SC_KERNELS:
        _SC_KERNELS["deg"] = _make_deg_kernel()
    return _SC_KERNELS["deg"](*args)


def _scatter(C, *args):
    if C not in _SC_KERNELS:
        _SC_KERNELS[C] = _make_scatter_kernel(C)
    return _SC_KERNELS[C](*args)


def _scatter1(*args):
    return _scatter(1, *args)


def _scatter2(*args):
    return _scatter(2, *args)


def _scatter4(*args):
    return _scatter(4, *args)


def kernel(x, edge_index, batch, W1, b1, W2, b2, W3, b3, Wf1, bf1, Wf2, bf2):
    src = edge_index[0]
    dst = edge_index[1]
    idx3 = jnp.stack([src.reshape(EB, 128), dst.reshape(EB, 128)], axis=1)
    zeros_n128 = jnp.zeros((N, 128), f32)
    ones_n128 = jnp.ones((128, 128), f32)
    batchcol = batch.reshape(N, 1)

    degp = _deg_kernel(idx3, ones_n128, zeros_n128)
    y1raw, counts = _tck1a(x, batchcol, W1)
    y1, dinv = _tck1b(y1raw, degp[0], degp[1])

    z1 = _scatter1(idx3, y1, zeros_n128)
    y20, y21 = _tck2(z1[0], z1[1], y1, dinv, b1.reshape(1, 128), W2)

    z2 = _scatter2(idx3, y20, y21, zeros_n128)
    y3c = _tck3(z2[0], z2[2], z2[1], z2[3], y20, y21, dinv,
                b2.reshape(1, 256), W3)

    z3 = _scatter4(idx3, *y3c, zeros_n128)
    seg = jnp.stack([batch[0::BLK], batch[BLK - 1::BLK]], axis=1).reshape(-1)
    counts2 = counts.reshape(G, 1)
    out = _tck45(seg, [z3[c] for c in range(8)], list(y3c), dinv,
                 b3.reshape(1, 512), batchcol, counts2,
                 Wf1, bf1.reshape(1, 1024), Wf2, bf2.reshape(1, 128))
    return out
